# baseline (speedup 1.0000x reference)
import functools

import jax
import jax.numpy as jnp
from jax.experimental import pallas as pl
from jax.experimental.pallas import tpu as pltpu


def _graphsage_body(adj_ref, x_ref, wt_ref, b_ref, out_ref):
    bm = adj_ref.shape[0]
    agg = jax.lax.dot_general(
        adj_ref[...], x_ref[...],
        dimension_numbers=(((1,), (0,)), ((), ())),
        precision=jax.lax.Precision.DEFAULT,
        preferred_element_type=jnp.float32)
    i = pl.program_id(0)
    agg = agg + x_ref[pl.ds(i * bm, bm), :]
    h = jax.lax.dot_general(
        agg, wt_ref[...],
        dimension_numbers=(((1,), (0,)), ((), ())),
        precision=jax.lax.Precision.DEFAULT,
        preferred_element_type=jnp.float32)
    h = jnp.maximum(h + b_ref[...], 0.0)
    norm = jnp.sqrt(jnp.sum(h * h, axis=1, keepdims=True))
    out_ref[...] = h / (norm + 1e-07)


@functools.partial(jax.jit, static_argnames=("block_m",))
def _graphsage(x, adj, W, b, block_m):
    n, d_in = x.shape
    d_out = W.shape[0]
    wt = W.T
    b2 = b.reshape(1, d_out)
    grid = (pl.cdiv(n, block_m),)
    return pl.pallas_call(
        _graphsage_body,
        grid=grid,
        in_specs=[
            pl.BlockSpec((block_m, n), lambda i: (i, 0)),
            pl.BlockSpec((n, d_in), lambda i: (0, 0)),
            pl.BlockSpec((d_in, d_out), lambda i: (0, 0)),
            pl.BlockSpec((1, d_out), lambda i: (0, 0)),
        ],
        out_specs=pl.BlockSpec((block_m, d_out), lambda i: (i, 0)),
        out_shape=jax.ShapeDtypeStruct((n, d_out), jnp.float32),
        compiler_params=pltpu.CompilerParams(
            dimension_semantics=("parallel",),
        ),
    )(adj, x, wt, b2)


def kernel(x, adj, W, b):
    return _graphsage(x, adj, W, b, block_m=400)
